# Initial kernel scaffold; baseline (speedup 1.0000x reference)
import jax
import jax.numpy as jnp
from jax.experimental import pallas as pl


def kernel(input_ids, embed_table, prompt_weight):
    raise NotImplementedError("write your pallas kernel here")


# speedup vs baseline: 6.3475x; 6.3475x over previous
import functools

import jax
import jax.numpy as jnp
from jax import lax
from jax.experimental import pallas as pl
from jax.experimental.pallas import tpu as pltpu
from jax.experimental.pallas import tpu_sc as plsc

B, S, P, D = 4096, 50, 20, 128
V = 100000
NC, NS = 2, 16
NW = NC * NS
BPW = B // NW
R = 4
NCHUNK = BPW // R

_mesh = plsc.VectorSubcoreMesh(
    core_axis_name="c", subcore_axis_name="s", num_cores=NC, num_subcores=NS
)


@functools.partial(
    pl.kernel,
    out_type=jax.ShapeDtypeStruct((B, P + S, D), jnp.float32),
    mesh=_mesh,
    scratch_types=[
        pltpu.VMEM((BPW, S), jnp.int32),
        pltpu.VMEM((R, P + S, D), jnp.float32),
        pltpu.SemaphoreType.DMA,
    ],
)
def _prompt_embed(ids_hbm, table_hbm, prompt_hbm, out_hbm, idx_v, buf, gsem):
    wid = lax.axis_index("s") * NC + lax.axis_index("c")
    row0 = wid * BPW

    pltpu.sync_copy(ids_hbm.at[pl.ds(row0, BPW)], idx_v)
    for r in range(R):
        pltpu.sync_copy(prompt_hbm, buf.at[r, pl.ds(0, P)])

    def chunk_body(c, carry):
        base = c * R
        copies = [
            pltpu.async_copy(
                table_hbm.at[idx_v.at[base + r]],
                buf.at[r, pl.ds(P, S)],
                gsem,
            )
            for r in range(R)
        ]
        for cp in copies:
            cp.wait()
        pltpu.sync_copy(buf, out_hbm.at[pl.ds(row0 + base, R)])
        return carry

    lax.fori_loop(0, NCHUNK, chunk_body, None)


def kernel(input_ids, embed_table, prompt_weight):
    return _prompt_embed(input_ids.astype(jnp.int32), embed_table, prompt_weight)


# speedup vs baseline: 6.7696x; 1.0665x over previous
import functools

import jax
import jax.numpy as jnp
from jax import lax
from jax.experimental import pallas as pl
from jax.experimental.pallas import tpu as pltpu
from jax.experimental.pallas import tpu_sc as plsc

B, S, P, D = 4096, 50, 20, 128
V = 100000
NC, NS = 2, 16
NW = NC * NS
BPW = B // NW
R = 4
NCHUNK = BPW // R

_mesh = plsc.VectorSubcoreMesh(
    core_axis_name="c", subcore_axis_name="s", num_cores=NC, num_subcores=NS
)


@functools.partial(
    pl.kernel,
    out_type=jax.ShapeDtypeStruct((B, P + S, D), jnp.float32),
    mesh=_mesh,
    scratch_types=[
        pltpu.VMEM((BPW, S), jnp.int32),
        pltpu.VMEM((2, R, P + S, D), jnp.float32),
        pltpu.SemaphoreType.DMA,
        pltpu.SemaphoreType.DMA,
        pltpu.SemaphoreType.DMA,
        pltpu.SemaphoreType.DMA,
    ],
)
def _prompt_embed(ids_hbm, table_hbm, prompt_hbm, out_hbm, idx_v, buf,
                  gsem0, gsem1, ssem0, ssem1):
    wid = lax.axis_index("s") * NC + lax.axis_index("c")
    row0 = wid * BPW
    gsem = (gsem0, gsem1)
    ssem = (ssem0, ssem1)

    pltpu.sync_copy(ids_hbm.at[pl.ds(row0, BPW)], idx_v)
    for slot in range(2):
        for r in range(R):
            pltpu.sync_copy(prompt_hbm, buf.at[slot, r, pl.ds(0, P)])

    def gather_copies(c, slot):
        base = c * R
        return [
            pltpu.make_async_copy(
                table_hbm.at[idx_v.at[base + r]],
                buf.at[slot, r, pl.ds(P, S)],
                gsem[slot],
            )
            for r in range(R)
        ]

    def store_copy(c, slot):
        return pltpu.make_async_copy(
            buf.at[slot], out_hbm.at[pl.ds(row0 + c * R, R)], ssem[slot]
        )

    for cp in gather_copies(0, 0):
        cp.start()

    def pair_body(p, carry):
        c0 = 2 * p
        @pl.when(p > 0)
        def _():
            store_copy(c0 - 1, 1).wait()
        for cp in gather_copies(c0 + 1, 1):
            cp.start()
        for cp in gather_copies(c0, 0):
            cp.wait()
        store_copy(c0, 0).start()

        store_copy(c0, 0).wait()
        @pl.when(p + 1 < NCHUNK // 2)
        def _():
            for cp in gather_copies(c0 + 2, 0):
                cp.start()
        for cp in gather_copies(c0 + 1, 1):
            cp.wait()
        store_copy(c0 + 1, 1).start()
        return carry

    lax.fori_loop(0, NCHUNK // 2, pair_body, None)
    store_copy(NCHUNK - 1, 1).wait()


def kernel(input_ids, embed_table, prompt_weight):
    return _prompt_embed(input_ids.astype(jnp.int32), embed_table, prompt_weight)


# speedup vs baseline: 13.0871x; 1.9332x over previous
import functools

import jax
import jax.numpy as jnp
from jax import lax
from jax.experimental import pallas as pl
from jax.experimental.pallas import tpu as pltpu
from jax.experimental.pallas import tpu_sc as plsc

B, S, P, D = 4096, 50, 20, 128
NC, NS = 2, 16
NW = NC * NS
BPW = B // NW
NPAIR = S // 2

_mesh = plsc.VectorSubcoreMesh(
    core_axis_name="c", subcore_axis_name="s", num_cores=NC, num_subcores=NS
)


@functools.partial(
    pl.kernel,
    out_type=jax.ShapeDtypeStruct((P + S, B, D), jnp.float32),
    mesh=_mesh,
    scratch_types=[
        pltpu.VMEM((S, BPW), jnp.int32),
        pltpu.VMEM((2, BPW, D), jnp.float32),
        pltpu.VMEM((P, D), jnp.float32),
        pltpu.VMEM((16, D), jnp.float32),
        pltpu.VMEM_SHARED((P, BPW, D), jnp.float32),
        pltpu.SemaphoreType.DMA,
        pltpu.SemaphoreType.DMA,
        pltpu.SemaphoreType.DMA,
        pltpu.SemaphoreType.DMA,
        pltpu.SemaphoreType.DMA,
    ],
)
def _prompt_embed(ids_hbm, table_hbm, prompt_hbm, out_hbm, idx_v, gbuf,
                  prompt_v, brep, shared, gsem0, gsem1, ssem0, ssem1, psem):
    sid = lax.axis_index("s")
    wid = sid * NC + lax.axis_index("c")
    b0 = wid * BPW
    gsem = (gsem0, gsem1)
    ssem = (ssem0, ssem1)

    pltpu.sync_copy(ids_hbm.at[pl.ds(0, S), pl.ds(b0, BPW)], idx_v)
    pltpu.sync_copy(prompt_hbm, prompt_v)

    for rep in range(2):
        t = sid + NS * rep

        @pl.when(t < P)
        def _():
            for c in range(D // 16):
                v = prompt_v[t, pl.ds(16 * c, 16)]
                for j in range(16):
                    brep[j, pl.ds(16 * c, 16)] = v
            for k in range(BPW // 16):
                pltpu.sync_copy(brep, shared.at[t, pl.ds(16 * k, 16)])

    plsc.subcore_barrier()

    pcopies = [
        pltpu.make_async_copy(
            shared.at[t], out_hbm.at[t, pl.ds(b0, BPW)], psem
        )
        for t in range(P)
    ]
    for cp in pcopies:
        cp.start()

    def g_copy(s, slot):
        return pltpu.make_async_copy(
            table_hbm.at[idx_v.at[s]], gbuf.at[slot], gsem[slot]
        )

    def s_copy(s, slot):
        return pltpu.make_async_copy(
            gbuf.at[slot], out_hbm.at[P + s, pl.ds(b0, BPW)], ssem[slot]
        )

    g_copy(0, 0).start()

    def pair_body(p, carry):
        s0 = 2 * p

        @pl.when(p > 0)
        def _():
            s_copy(s0 - 1, 1).wait()
        g_copy(s0 + 1, 1).start()
        g_copy(s0, 0).wait()
        s_copy(s0, 0).start()

        s_copy(s0, 0).wait()

        @pl.when(p + 1 < NPAIR)
        def _():
            g_copy(s0 + 2, 0).start()
        g_copy(s0 + 1, 1).wait()
        s_copy(s0 + 1, 1).start()
        return carry

    lax.fori_loop(0, NPAIR, pair_body, None)
    s_copy(S - 1, 1).wait()
    for cp in pcopies:
        cp.wait()


def kernel(input_ids, embed_table, prompt_weight):
    ids_t = jnp.transpose(input_ids.astype(jnp.int32))
    out_t = _prompt_embed(ids_t, embed_table, prompt_weight)
    return jnp.transpose(out_t, (1, 0, 2))


# speedup vs baseline: 13.2949x; 1.0159x over previous
import functools

import jax
import jax.numpy as jnp
from jax import lax
from jax.experimental import pallas as pl
from jax.experimental.pallas import tpu as pltpu
from jax.experimental.pallas import tpu_sc as plsc

B, S, P, D = 4096, 50, 20, 128
NC, NS = 2, 16
NW = NC * NS
BPW = B // NW
NBUF = 5
AHEAD = 2
NGROUP = S // NBUF

_mesh = plsc.VectorSubcoreMesh(
    core_axis_name="c", subcore_axis_name="s", num_cores=NC, num_subcores=NS
)


@functools.partial(
    pl.kernel,
    out_type=jax.ShapeDtypeStruct((P + S, B, D), jnp.float32),
    mesh=_mesh,
    scratch_types=[
        pltpu.VMEM((S, BPW), jnp.int32),
        pltpu.VMEM((NBUF, BPW, D), jnp.float32),
        pltpu.VMEM((P, D), jnp.float32),
        pltpu.VMEM((16, D), jnp.float32),
        pltpu.VMEM_SHARED((P, BPW, D), jnp.float32),
        [pltpu.SemaphoreType.DMA] * NBUF,
        [pltpu.SemaphoreType.DMA] * NBUF,
        pltpu.SemaphoreType.DMA,
    ],
)
def _prompt_embed(ids_hbm, table_hbm, prompt_hbm, out_hbm, idx_v, gbuf,
                  prompt_v, brep, shared, gsem, ssem, psem):
    sid = lax.axis_index("s")
    wid = sid * NC + lax.axis_index("c")
    b0 = wid * BPW

    def g_copy(s, slot):
        return pltpu.make_async_copy(
            table_hbm.at[idx_v.at[s]], gbuf.at[slot], gsem[slot]
        )

    def s_copy(s, slot):
        return pltpu.make_async_copy(
            gbuf.at[slot], out_hbm.at[P + s, pl.ds(b0, BPW)], ssem[slot]
        )

    def p_copy(t):
        return pltpu.make_async_copy(
            shared.at[t], out_hbm.at[t, pl.ds(b0, BPW)], psem
        )

    pltpu.sync_copy(ids_hbm.at[pl.ds(0, S), pl.ds(b0, BPW)], idx_v)
    for s in range(AHEAD):
        g_copy(s, s).start()

    pltpu.sync_copy(prompt_hbm, prompt_v)

    for rep in range(2):
        t = sid + NS * rep

        @pl.when(t < P)
        def _():
            for c in range(D // 16):
                v = prompt_v[t, pl.ds(16 * c, 16)]
                for j in range(16):
                    brep[j, pl.ds(16 * c, 16)] = v
            for k in range(BPW // 16):
                pltpu.sync_copy(brep, shared.at[t, pl.ds(16 * k, 16)])

    plsc.subcore_barrier()

    def group_body(g, carry):
        for k in range(NBUF):
            s = NBUF * g + k
            g_copy(s, k).wait()
            s_copy(s, k).start()
            if k == 1 or k == 3:
                t = 2 * g + (k - 1) // 2
                p_copy(t).start()
            nxt = (k + AHEAD) % NBUF

            @pl.when(s + AHEAD < S)
            def _():
                @pl.when(s >= NBUF - AHEAD)
                def _():
                    s_copy(s - (NBUF - AHEAD), nxt).wait()
                g_copy(s + AHEAD, nxt).start()
        return carry

    lax.fori_loop(0, NGROUP, group_body, None)
    for k in range(NBUF):
        s_copy(S - NBUF + k, (S - NBUF + k) % NBUF).wait()
    for t in range(P):
        p_copy(t).wait()


def kernel(input_ids, embed_table, prompt_weight):
    ids_t = jnp.transpose(input_ids.astype(jnp.int32))
    out_t = _prompt_embed(ids_t, embed_table, prompt_weight)
    return jnp.transpose(out_t, (1, 0, 2))
